# baseline (speedup 1.0000x reference)
import functools

import jax
import jax.numpy as jnp
from jax import lax
from jax.experimental import pallas as pl
from jax.experimental.pallas import tpu as pltpu
from jax.experimental.pallas import tpu_sc as plsc

EMBED = 64
SCALE = 8.0
NW = 32
SUB = 128
CHUNK = 640
LANES = 16
UNROLL = 4
CHUNK_BYTES = CHUNK * EMBED * 4


@functools.lru_cache(maxsize=None)
def _build(B, V):
    BPW = B // NW
    NSUB = CHUNK // SUB
    NCHUNK = BPW // CHUNK
    assert NCHUNK * CHUNK == BPW

    mesh = plsc.VectorSubcoreMesh(core_axis_name="c", subcore_axis_name="s")

    @functools.partial(
        pl.kernel,
        mesh=mesh,
        out_type=jax.ShapeDtypeStruct((B, EMBED), jnp.float32),
        scratch_types=[
            pltpu.VMEM((BPW,), jnp.int32),
            pltpu.VMEM((2, CHUNK, EMBED), jnp.float32),
            pltpu.SemaphoreType.DMA,
            pltpu.SemaphoreType.DMA,
        ],
        compiler_params=pltpu.CompilerParams(use_tc_tiling_on_sc=False),
    )
    def k(idx_hbm, lut_hbm, out_hbm, idx_v, rows_v, gsem, ssem):
        wid = lax.axis_index("s") * 2 + lax.axis_index("c")
        base = wid * BPW
        pltpu.sync_copy(idx_hbm.at[pl.ds(base, BPW)], idx_v)

        def fire_gathers(c, buf):
            for j in range(NSUB):
                pltpu.async_copy(
                    lut_hbm.at[idx_v.at[pl.ds(c * CHUNK + j * SUB, SUB)]],
                    rows_v.at[buf, pl.ds(j * SUB, SUB)],
                    gsem,
                )

        def drain(sem, buf):
            pltpu.make_async_copy(
                lut_hbm.at[pl.ds(0, CHUNK)], rows_v.at[buf], sem).wait()

        fire_gathers(0, 0)

        def chunk_body(c, carry):
            cur = c % 2
            nxt = 1 - cur
            drain(gsem, cur)

            @pl.when(c >= 1)
            def _():
                drain(ssem, nxt)

            @pl.when(c + 1 < NCHUNK)
            def _():
                fire_gathers(c + 1, nxt)

            def scale_rows(r, carry2):
                for u in range(UNROLL):
                    row = r * UNROLL + u
                    for j in range(EMBED // LANES):
                        sl = pl.ds(j * LANES, LANES)
                        rows_v[cur, row, sl] = rows_v[cur, row, sl] * SCALE
                return carry2

            lax.fori_loop(0, CHUNK // UNROLL, scale_rows, 0)
            pltpu.async_copy(
                rows_v.at[cur], out_hbm.at[pl.ds(base + c * CHUNK, CHUNK)],
                ssem)
            return carry

        lax.fori_loop(0, NCHUNK, chunk_body, 0)
        drain(ssem, (NCHUNK - 1) % 2)

    return k


def kernel(x, lut):
    B = x.shape[0] * x.shape[1]
    xf = x.reshape(B).astype(jnp.int32)
    out = _build(B, lut.shape[0])(xf, lut)
    return out.reshape(x.shape[0], x.shape[1], EMBED)


# speedup vs baseline: 1.3610x; 1.3610x over previous
import functools

import jax
import jax.numpy as jnp
from jax import lax
from jax.experimental import pallas as pl
from jax.experimental.pallas import tpu as pltpu
from jax.experimental.pallas import tpu_sc as plsc

EMBED = 64
SCALE = 8.0
NW = 32
BCHUNK = 16
LANES = 16
UNROLL = 4


@functools.lru_cache(maxsize=None)
def _build(BATCH, HIST, V):
    BPW = (BATCH // NW) * HIST
    CHUNK = BCHUNK * HIST
    NCHUNK = BPW // CHUNK
    assert NCHUNK * CHUNK == BPW
    subs = []
    off = 0
    while off < CHUNK:
        n = min(128, CHUNK - off)
        subs.append((off, n))
        off += n

    mesh = plsc.VectorSubcoreMesh(core_axis_name="c", subcore_axis_name="s")

    @functools.partial(
        pl.kernel,
        mesh=mesh,
        out_type=jax.ShapeDtypeStruct((BATCH, HIST, EMBED), jnp.float32),
        scratch_types=[
            pltpu.VMEM((BPW,), jnp.int32),
            pltpu.VMEM((CHUNK, EMBED), jnp.float32),
            pltpu.SemaphoreType.DMA,
        ],
        compiler_params=pltpu.CompilerParams(use_tc_tiling_on_sc=False),
    )
    def k(idx_hbm, lut_hbm, out_hbm, idx_v, rows_v, gsem):
        wid = lax.axis_index("s") * 2 + lax.axis_index("c")
        pltpu.sync_copy(idx_hbm.at[pl.ds(wid * BPW, BPW)], idx_v)
        b0 = wid * (BATCH // NW)

        def chunk_body(c, carry):
            copies = []
            for (soff, n) in subs:
                copies.append(pltpu.async_copy(
                    lut_hbm.at[idx_v.at[pl.ds(c * CHUNK + soff, n)]],
                    rows_v.at[pl.ds(soff, n)],
                    gsem,
                ))
            for cp in copies:
                cp.wait()

            def scale_rows(r, carry2):
                for u in range(UNROLL):
                    row = r * UNROLL + u
                    for j in range(EMBED // LANES):
                        sl = pl.ds(j * LANES, LANES)
                        rows_v[row, sl] = rows_v[row, sl] * SCALE
                return carry2

            lax.fori_loop(0, CHUNK // UNROLL, scale_rows, 0)
            outs = []
            for i in range(BCHUNK):
                outs.append(pltpu.async_copy(
                    rows_v.at[pl.ds(i * HIST, HIST)],
                    out_hbm.at[b0 + c * BCHUNK + i],
                    gsem,
                ))
            for cp in outs:
                cp.wait()
            return carry

        lax.fori_loop(0, NCHUNK, chunk_body, 0)

    return k


def kernel(x, lut):
    BATCH, HIST = x.shape
    xf = x.reshape(BATCH * HIST).astype(jnp.int32)
    return _build(BATCH, HIST, lut.shape[0])(xf, lut)
